# baseline (speedup 1.0000x reference)
import functools

import jax
import jax.numpy as jnp
import numpy as np
from jax import lax
from jax.experimental import pallas as pl
from jax.experimental.pallas import tpu as pltpu
from jax.experimental.pallas import tpu_sc as plsc

N_BINS = 15
N_ROWS = 50000
N_COLS = 1000
COL_BLOCK = 4608
GRID = (N_ROWS + COL_BLOCK - 1) // COL_BLOCK
N_PAD = GRID * COL_BLOCK

NC = 2
NS = 16
NW = NC * NS
CHUNK = N_PAD // NW
N_ACC = 3 * N_BINS

_BOUNDS = np.array(
    [0x00000000, 0x3D888889, 0x3E088889, 0x3E4CCCCE, 0x3E888889, 0x3EAAAAAB,
     0x3ECCCCCE, 0x3EEEEEF0, 0x3F088889, 0x3F19999A, 0x3F2AAAAB, 0x3F3BBBBC,
     0x3F4CCCCE, 0x3F5DDDDF, 0x3F6EEEF0, 0x3F800000],
    dtype=np.uint32,
).view(np.float32)


def _conf_body(block_off, x_ref, lbl_ref, conf_ref, acc_ref):
    step = pl.program_id(0) + block_off
    x = x_ref[...]
    m = jnp.max(x, axis=0, keepdims=True)
    s = jnp.sum(jnp.exp(x - m), axis=0, keepdims=True)
    conf = 1.0 / s

    ii = lax.broadcasted_iota(jnp.int32, x.shape, 0)
    pred = jnp.min(jnp.where(x == m, ii, N_COLS), axis=0, keepdims=True)
    acc = (pred == lbl_ref[...]).astype(jnp.float32)

    sidx = step * COL_BLOCK + lax.broadcasted_iota(jnp.int32, (1, COL_BLOCK), 1)
    valid = sidx < N_ROWS
    conf_ref[...] = jnp.where(valid, conf, 0.0)
    acc_ref[...] = jnp.where(valid, acc, 0.0)


def _sc_bin_body(chunk, conf_hbm, acc_hbm, out_hbm, conf_v, acc_v, bins_v,
                 tmp_v, shared):
    cid = lax.axis_index("c")
    sid = lax.axis_index("s")
    wid = sid * NC + cid
    base = wid * chunk
    pltpu.sync_copy(conf_hbm.at[pl.ds(base, chunk)], conf_v)
    pltpu.sync_copy(acc_hbm.at[pl.ds(base, chunk)], acc_v)

    zero = jnp.zeros((16,), jnp.float32)

    def chunk_step(i, carry):
        off = pl.multiple_of(i * 16, 16)
        c = conf_v[pl.ds(off, 16)]
        a = acc_v[pl.ds(off, 16)]
        out = []
        for b in range(N_BINS):
            inb = (c > _BOUNDS[b]) & (c <= _BOUNDS[b + 1])
            cnt, cs, asum = carry[b]
            out.append((
                cnt + jnp.where(inb, 1.0, zero),
                cs + jnp.where(inb, c, zero),
                asum + jnp.where(inb, a, zero),
            ))
        return tuple(out)

    init = tuple((zero, zero, zero) for _ in range(N_BINS))
    acc_bins = lax.fori_loop(0, chunk // 16, chunk_step, init)
    for b in range(N_BINS):
        cnt, cs, asum = acc_bins[b]
        bins_v[3 * b + 0, :] = cnt
        bins_v[3 * b + 1, :] = cs
        bins_v[3 * b + 2, :] = asum

    pltpu.sync_copy(bins_v, shared.at[sid])
    plsc.subcore_barrier()

    @pl.when(sid == 0)
    def _writeout():
        for w in range(1, NS):
            pltpu.sync_copy(shared.at[w], tmp_v)
            for r in range(N_ACC):
                bins_v[r, :] += tmp_v[r, :]
        pltpu.sync_copy(bins_v, out_hbm.at[cid])


@functools.lru_cache(maxsize=4)
def _make_sc_bin(n_samples):
    chunk = n_samples // NW
    mesh = plsc.VectorSubcoreMesh(core_axis_name="c", subcore_axis_name="s")
    return pl.kernel(
        functools.partial(_sc_bin_body, chunk),
        mesh=mesh,
        out_type=jax.ShapeDtypeStruct((NC, N_ACC, 16), jnp.float32),
        scratch_types=[
            pltpu.VMEM((chunk,), jnp.float32),
            pltpu.VMEM((chunk,), jnp.float32),
            pltpu.VMEM((N_ACC, 16), jnp.float32),
            pltpu.VMEM((N_ACC, 16), jnp.float32),
            pltpu.VMEM_SHARED((NS, N_ACC, 16), jnp.float32),
        ],
    )


def _tc_stage(xt, lbl, nblocks, block_off):
    n_out = nblocks * COL_BLOCK
    return pl.pallas_call(
        functools.partial(_conf_body, block_off),
        grid=(nblocks,),
        in_specs=[
            pl.BlockSpec((N_COLS, COL_BLOCK), lambda i: (0, i + block_off)),
            pl.BlockSpec((1, COL_BLOCK), lambda i: (0, i + block_off)),
        ],
        out_specs=[
            pl.BlockSpec((1, COL_BLOCK), lambda i: (0, i)),
            pl.BlockSpec((1, COL_BLOCK), lambda i: (0, i)),
        ],
        out_shape=[
            jax.ShapeDtypeStruct((1, n_out), jnp.float32),
            jax.ShapeDtypeStruct((1, n_out), jnp.float32),
        ],
        compiler_params=pltpu.CompilerParams(
            dimension_semantics=("arbitrary",),
        ),
    )(xt, lbl)


H1_BLOCKS = 6
H2_BLOCKS = GRID - H1_BLOCKS


@jax.jit
def kernel(logits, labels):
    xt = logits.T
    lbl = labels.astype(jnp.int32).reshape(1, N_ROWS)
    conf1, acc1 = _tc_stage(xt, lbl, H1_BLOCKS, 0)
    n1 = H1_BLOCKS * COL_BLOCK
    bins1 = _make_sc_bin(n1)(conf1.reshape(n1), acc1.reshape(n1))
    conf2, acc2 = _tc_stage(xt, lbl, H2_BLOCKS, H1_BLOCKS)
    n2 = H2_BLOCKS * COL_BLOCK
    bins2 = _make_sc_bin(n2)(conf2.reshape(n2), acc2.reshape(n2))
    tot = jnp.sum(bins1 + bins2, axis=(0, 2))
    cnt, cs, asum = tot[0::3], tot[1::3], tot[2::3]
    safe = jnp.maximum(cnt, 1.0)
    contrib = jnp.abs(cs / safe - asum / safe) * (cnt * (1.0 / N_ROWS))
    ece = jnp.sum(jnp.where(cnt > 0.0, contrib, 0.0))
    return ece.reshape(1)


# speedup vs baseline: 1.0208x; 1.0208x over previous
import functools

import jax
import jax.numpy as jnp
import numpy as np
from jax import lax
from jax.experimental import pallas as pl
from jax.experimental.pallas import tpu as pltpu
from jax.experimental.pallas import tpu_sc as plsc

N_BINS = 15
N_ROWS = 50000
N_COLS = 1000
COL_BLOCK = 4608
GRID = (N_ROWS + COL_BLOCK - 1) // COL_BLOCK
N_PAD = GRID * COL_BLOCK

NC = 2
NS = 16
NW = NC * NS
CHUNK = N_PAD // NW
N_ACC = 3 * N_BINS

_BOUNDS = np.array(
    [0x00000000, 0x3D888889, 0x3E088889, 0x3E4CCCCE, 0x3E888889, 0x3EAAAAAB,
     0x3ECCCCCE, 0x3EEEEEF0, 0x3F088889, 0x3F19999A, 0x3F2AAAAB, 0x3F3BBBBC,
     0x3F4CCCCE, 0x3F5DDDDF, 0x3F6EEEF0, 0x3F800000],
    dtype=np.uint32,
).view(np.float32)


def _conf_body(block_off, x_ref, lbl_ref, conf_ref, acc_ref):
    step = pl.program_id(0) + block_off
    x = x_ref[...]
    m = jnp.max(x, axis=0, keepdims=True)
    s = jnp.sum(jnp.exp(x - m), axis=0, keepdims=True)
    conf = 1.0 / s

    ii = lax.broadcasted_iota(jnp.int32, x.shape, 0)
    pred = jnp.min(jnp.where(x == m, ii, N_COLS), axis=0, keepdims=True)
    acc = (pred == lbl_ref[...]).astype(jnp.float32)

    sidx = step * COL_BLOCK + lax.broadcasted_iota(jnp.int32, (1, COL_BLOCK), 1)
    valid = sidx < N_ROWS
    conf_ref[...] = jnp.where(valid, conf, 0.0)
    acc_ref[...] = jnp.where(valid, acc, 0.0)


def _sc_bin_body(chunk, conf_hbm, acc_hbm, out_hbm, conf_v, acc_v, bins_v,
                 tmp_v, shared):
    cid = lax.axis_index("c")
    sid = lax.axis_index("s")
    wid = sid * NC + cid
    base = wid * chunk
    pltpu.sync_copy(conf_hbm.at[pl.ds(base, chunk)], conf_v)
    pltpu.sync_copy(acc_hbm.at[pl.ds(base, chunk)], acc_v)

    zero = jnp.zeros((16,), jnp.float32)

    def chunk_step(i, carry):
        off = pl.multiple_of(i * 16, 16)
        c = conf_v[pl.ds(off, 16)]
        a = acc_v[pl.ds(off, 16)]
        out = []
        for b in range(N_BINS):
            inb = (c > _BOUNDS[b]) & (c <= _BOUNDS[b + 1])
            cnt, cs, asum = carry[b]
            out.append((
                cnt + jnp.where(inb, 1.0, zero),
                cs + jnp.where(inb, c, zero),
                asum + jnp.where(inb, a, zero),
            ))
        return tuple(out)

    init = tuple((zero, zero, zero) for _ in range(N_BINS))
    acc_bins = lax.fori_loop(0, chunk // 16, chunk_step, init)
    for b in range(N_BINS):
        cnt, cs, asum = acc_bins[b]
        bins_v[3 * b + 0, :] = cnt
        bins_v[3 * b + 1, :] = cs
        bins_v[3 * b + 2, :] = asum

    pltpu.sync_copy(bins_v, shared.at[sid])
    plsc.subcore_barrier()

    @pl.when(sid == 0)
    def _writeout():
        for w in range(1, NS):
            pltpu.sync_copy(shared.at[w], tmp_v)
            for r in range(N_ACC):
                bins_v[r, :] += tmp_v[r, :]
        pltpu.sync_copy(bins_v, out_hbm.at[cid])


@functools.lru_cache(maxsize=4)
def _make_sc_bin(n_samples):
    chunk = n_samples // NW
    mesh = plsc.VectorSubcoreMesh(core_axis_name="c", subcore_axis_name="s")
    return pl.kernel(
        functools.partial(_sc_bin_body, chunk),
        mesh=mesh,
        out_type=jax.ShapeDtypeStruct((NC, N_ACC, 16), jnp.float32),
        scratch_types=[
            pltpu.VMEM((chunk,), jnp.float32),
            pltpu.VMEM((chunk,), jnp.float32),
            pltpu.VMEM((N_ACC, 16), jnp.float32),
            pltpu.VMEM((N_ACC, 16), jnp.float32),
            pltpu.VMEM_SHARED((NS, N_ACC, 16), jnp.float32),
        ],
    )


def _tc_stage(xt, lbl, nblocks, block_off):
    n_out = nblocks * COL_BLOCK
    return pl.pallas_call(
        functools.partial(_conf_body, block_off),
        grid=(nblocks,),
        in_specs=[
            pl.BlockSpec((N_COLS, COL_BLOCK), lambda i: (0, i + block_off)),
            pl.BlockSpec((1, COL_BLOCK), lambda i: (0, i + block_off)),
        ],
        out_specs=[
            pl.BlockSpec((1, COL_BLOCK), lambda i: (0, i)),
            pl.BlockSpec((1, COL_BLOCK), lambda i: (0, i)),
        ],
        out_shape=[
            jax.ShapeDtypeStruct((1, n_out), jnp.float32),
            jax.ShapeDtypeStruct((1, n_out), jnp.float32),
        ],
        compiler_params=pltpu.CompilerParams(
            dimension_semantics=("arbitrary",),
        ),
    )(xt, lbl)


@jax.jit
def kernel(logits, labels):
    xt = logits.T
    lbl = labels.astype(jnp.int32).reshape(1, N_ROWS)
    conf_row, acc_row = _tc_stage(xt, lbl, GRID, 0)
    bins = _make_sc_bin(N_PAD)(
        conf_row.reshape(N_PAD), acc_row.reshape(N_PAD)
    )
    tot = jnp.sum(bins, axis=(0, 2))
    cnt, cs, asum = tot[0::3], tot[1::3], tot[2::3]
    safe = jnp.maximum(cnt, 1.0)
    contrib = jnp.abs(cs / safe - asum / safe) * (cnt * (1.0 / N_ROWS))
    ece = jnp.sum(jnp.where(cnt > 0.0, contrib, 0.0))
    return ece.reshape(1)


# speedup vs baseline: 1.0254x; 1.0046x over previous
import functools

import jax
import jax.numpy as jnp
import numpy as np
from jax import lax
from jax.experimental import pallas as pl
from jax.experimental.pallas import tpu as pltpu
from jax.experimental.pallas import tpu_sc as plsc

N_BINS = 15
N_ROWS = 50000
N_COLS = 1000
COL_BLOCK = 4096
GRID = (N_ROWS + COL_BLOCK - 1) // COL_BLOCK
N_PAD = GRID * COL_BLOCK

NC = 2
NS = 16
NW = NC * NS
CHUNK = N_PAD // NW
N_ACC = 3 * N_BINS

_BOUNDS = np.array(
    [0x00000000, 0x3D888889, 0x3E088889, 0x3E4CCCCE, 0x3E888889, 0x3EAAAAAB,
     0x3ECCCCCE, 0x3EEEEEF0, 0x3F088889, 0x3F19999A, 0x3F2AAAAB, 0x3F3BBBBC,
     0x3F4CCCCE, 0x3F5DDDDF, 0x3F6EEEF0, 0x3F800000],
    dtype=np.uint32,
).view(np.float32)


def _conf_body(block_off, x_ref, lbl_ref, conf_ref, acc_ref):
    step = pl.program_id(0) + block_off
    x = x_ref[...]
    m = jnp.max(x, axis=0, keepdims=True)
    s = jnp.sum(jnp.exp(x - m), axis=0, keepdims=True)
    conf = 1.0 / s

    ii = lax.broadcasted_iota(jnp.int32, x.shape, 0)
    pred = jnp.min(jnp.where(x == m, ii, N_COLS), axis=0, keepdims=True)
    acc = (pred == lbl_ref[...]).astype(jnp.float32)

    sidx = step * COL_BLOCK + lax.broadcasted_iota(jnp.int32, (1, COL_BLOCK), 1)
    valid = sidx < N_ROWS
    conf_ref[...] = jnp.where(valid, conf, 0.0)
    acc_ref[...] = jnp.where(valid, acc, 0.0)


def _sc_bin_body(chunk, conf_hbm, acc_hbm, out_hbm, conf_v, acc_v, bins_v,
                 tmp_v, shared):
    cid = lax.axis_index("c")
    sid = lax.axis_index("s")
    wid = sid * NC + cid
    base = wid * chunk
    pltpu.sync_copy(conf_hbm.at[0, pl.ds(base, chunk)], conf_v)
    pltpu.sync_copy(acc_hbm.at[0, pl.ds(base, chunk)], acc_v)

    zero = jnp.zeros((16,), jnp.float32)

    def chunk_step(i, carry):
        off = pl.multiple_of(i * 16, 16)
        c = conf_v[pl.ds(off, 16)]
        a = acc_v[pl.ds(off, 16)]
        out = []
        for b in range(N_BINS):
            inb = (c > _BOUNDS[b]) & (c <= _BOUNDS[b + 1])
            cnt, cs, asum = carry[b]
            out.append((
                cnt + jnp.where(inb, 1.0, zero),
                cs + jnp.where(inb, c, zero),
                asum + jnp.where(inb, a, zero),
            ))
        return tuple(out)

    init = tuple((zero, zero, zero) for _ in range(N_BINS))
    acc_bins = lax.fori_loop(0, chunk // 16, chunk_step, init)
    for b in range(N_BINS):
        cnt, cs, asum = acc_bins[b]
        bins_v[3 * b + 0, :] = cnt
        bins_v[3 * b + 1, :] = cs
        bins_v[3 * b + 2, :] = asum

    pltpu.sync_copy(bins_v, shared.at[sid])
    plsc.subcore_barrier()

    @pl.when(sid == 0)
    def _writeout():
        for w in range(1, NS):
            pltpu.sync_copy(shared.at[w], tmp_v)
            for r in range(N_ACC):
                bins_v[r, :] += tmp_v[r, :]
        pltpu.sync_copy(bins_v, out_hbm.at[cid])


@functools.lru_cache(maxsize=4)
def _make_sc_bin(n_samples):
    chunk = n_samples // NW
    mesh = plsc.VectorSubcoreMesh(core_axis_name="c", subcore_axis_name="s")
    return pl.kernel(
        functools.partial(_sc_bin_body, chunk),
        mesh=mesh,
        out_type=jax.ShapeDtypeStruct((NC, N_ACC, 16), jnp.float32),
        scratch_types=[
            pltpu.VMEM((chunk,), jnp.float32),
            pltpu.VMEM((chunk,), jnp.float32),
            pltpu.VMEM((N_ACC, 16), jnp.float32),
            pltpu.VMEM((N_ACC, 16), jnp.float32),
            pltpu.VMEM_SHARED((NS, N_ACC, 16), jnp.float32),
        ],
    )


def _tc_stage(xt, lbl, nblocks, block_off):
    n_out = nblocks * COL_BLOCK
    return pl.pallas_call(
        functools.partial(_conf_body, block_off),
        grid=(nblocks,),
        in_specs=[
            pl.BlockSpec((N_COLS, COL_BLOCK), lambda i: (0, i + block_off)),
            pl.BlockSpec((1, COL_BLOCK), lambda i: (0, i + block_off)),
        ],
        out_specs=[
            pl.BlockSpec((1, COL_BLOCK), lambda i: (0, i)),
            pl.BlockSpec((1, COL_BLOCK), lambda i: (0, i)),
        ],
        out_shape=[
            jax.ShapeDtypeStruct((1, n_out), jnp.float32),
            jax.ShapeDtypeStruct((1, n_out), jnp.float32),
        ],
        compiler_params=pltpu.CompilerParams(
            dimension_semantics=("arbitrary",),
        ),
    )(xt, lbl)


@jax.jit
def kernel(logits, labels):
    xt = logits.T
    lbl = labels.astype(jnp.int32).reshape(1, N_ROWS)
    conf_row, acc_row = _tc_stage(xt, lbl, GRID, 0)
    bins = _make_sc_bin(N_PAD)(conf_row, acc_row)
    tot = jnp.sum(bins, axis=(0, 2))
    cnt, cs, asum = tot[0::3], tot[1::3], tot[2::3]
    safe = jnp.maximum(cnt, 1.0)
    contrib = jnp.abs(cs / safe - asum / safe) * (cnt * (1.0 / N_ROWS))
    ece = jnp.sum(jnp.where(cnt > 0.0, contrib, 0.0))
    return ece.reshape(1)
